# baseline (speedup 1.0000x reference)
import functools

import jax
import jax.numpy as jnp
import numpy as np
from jax import lax
from jax.experimental import pallas as pl
from jax.experimental.pallas import tpu as pltpu
from jax.experimental.pallas import tpu_sc as plsc

_G = 2
_R = 2
_DIM = 1024
_DPG = _DIM // _G
_CD = 4
_NIND = 625
_NBINS = 640
_EPS = 1e-5
_HALF_L = 4.0 * (1.0 + 1e-3) / 2.0
_TB = 2048
_TBB = 2048

_NW = 32
_CHUNK = 1024


def _quant_tc_body(x_ref, wi_ref, bi_ref, ind_ref, qb_ref):
    for g in range(_G):
        xb = x_ref[0, g * _DPG:(g + 1) * _DPG]
        h = jnp.dot(wi_ref[g], xb, preferred_element_type=jnp.float32)
        h = h + bi_ref[g]

        def stage(res, scale_inv, scale):
            q = jnp.round(jnp.tanh(res * scale_inv) * _HALF_L)
            idx = ((q[3] * 5.0 + q[2]) * 5.0 + q[1]) * 5.0 + q[0] + 312.0
            return q * (0.5 * scale), idx

        quant0, idx0 = stage(h, 1.0, 1.0)
        quant1, idx1 = stage(h - quant0, 4.0, 0.25)
        qout = quant0 + quant1

        ind_ref[0, g * _R] = idx0.astype(jnp.int32)
        ind_ref[0, g * _R + 1] = idx1.astype(jnp.int32)
        qb_ref[0, g * _CD:(g + 1) * _CD] = qout


def _quant_tc(x, w_in, b_in):
    b, _, t = x.shape
    grid = (b, t // _TB)
    out_shapes = (
        jax.ShapeDtypeStruct((b, _G * _R, t), jnp.int32),
        jax.ShapeDtypeStruct((b, _G * _CD, t), jnp.float32),
    )
    return pl.pallas_call(
        _quant_tc_body,
        grid=grid,
        in_specs=[
            pl.BlockSpec((1, _DIM, _TB), lambda bb, tt: (bb, 0, tt)),
            pl.BlockSpec((_G, _CD, _DPG), lambda bb, tt: (0, 0, 0)),
            pl.BlockSpec((_G, _CD, 1), lambda bb, tt: (0, 0, 0)),
        ],
        out_specs=(
            pl.BlockSpec((1, _G * _R, _TB), lambda bb, tt: (bb, 0, tt)),
            pl.BlockSpec((1, _G * _CD, _TB), lambda bb, tt: (bb, 0, tt)),
        ),
        out_shape=out_shapes,
        compiler_params=pltpu.CompilerParams(
            dimension_semantics=("arbitrary", "arbitrary"),
        ),
    )(x, w_in, b_in)


def _project_tc_body(qb_ref, wo_ref, bo_ref, feat_ref):
    for g in range(_G):
        wo = wo_ref[g]
        wh = wo.astype(jnp.bfloat16)
        wl = (wo - wh.astype(jnp.float32)).astype(jnp.bfloat16)
        qout = qb_ref[0, g * _CD:(g + 1) * _CD].astype(jnp.bfloat16)
        feat_ref[0, g * _DPG:(g + 1) * _DPG] = (
            jnp.dot(wh, qout, preferred_element_type=jnp.float32)
            + jnp.dot(wl, qout, preferred_element_type=jnp.float32)
            + bo_ref[g])


def _project_tc(qb, w_out, b_out3):
    b, _, t = qb.shape
    grid = (b, t // _TBB)
    return pl.pallas_call(
        _project_tc_body,
        grid=grid,
        in_specs=[
            pl.BlockSpec((1, _G * _CD, _TBB), lambda bb, tt: (bb, 0, tt)),
            pl.BlockSpec((_G, _DPG, _CD), lambda bb, tt: (0, 0, 0)),
            pl.BlockSpec((_G, _DPG, 1), lambda bb, tt: (0, 0, 0)),
        ],
        out_specs=pl.BlockSpec((1, _DIM, _TBB), lambda bb, tt: (bb, 0, tt)),
        out_shape=jax.ShapeDtypeStruct((b, _DIM, t), jnp.float32),
        compiler_params=pltpu.CompilerParams(
            dimension_semantics=("arbitrary", "arbitrary"),
        ),
    )(qb, w_out, b_out3)


def _hist_sc_body(ind_hbm, out_hbm, idx_v, hist_v, acc_v, stage_v, shared_v):
    c = lax.axis_index("c")
    s = lax.axis_index("s")
    w = c * 16 + s
    gr = w // 8
    j = w % 8
    b = j // 2
    hh = j % 2

    pltpu.sync_copy(ind_hbm.at[b, gr, pl.ds(hh * _CHUNK, _CHUNK)], idx_v)

    zeros16 = jnp.zeros((16,), dtype=jnp.float32)
    lanes = lax.iota(jnp.int32, 16)

    def zero_body(i, carry):
        for u in range(8):
            hist_v[pl.ds((i * 8 + u) * 16, 16)] = zeros16
        return carry

    lax.fori_loop(0, 16 * _NBINS // 16 // 8, zero_body, 0)

    ones16 = jnp.ones((16,), dtype=jnp.float32)
    lane_off = lanes * _NBINS

    def scat_body(i, carry):
        for u in range(8):
            v = idx_v[pl.ds((i * 8 + u) * 16, 16)]
            plsc.addupdate_scatter(hist_v, [lane_off + v], ones16)
        return carry

    lax.fori_loop(0, _CHUNK // 16 // 8, scat_body, 0)

    def red_body(j, carry):
        tot = hist_v[pl.ds(j * 16, 16)]
        for l in range(1, 16):
            tot = tot + hist_v[pl.ds(l * _NBINS + j * 16, 16)]
        acc_v[pl.ds(j * 16, 16)] = tot
        return carry

    lax.fori_loop(0, _NBINS // 16, red_body, 0)

    pltpu.sync_copy(acc_v, shared_v.at[s])
    plsc.subcore_barrier()

    @pl.when(jnp.logical_or(s == 0, s == 8))
    def _():
        base = (s // 8) * 8
        pltpu.sync_copy(shared_v.at[pl.ds(base, 8)], stage_v)

        inv = jnp.float32(1.0 / 8192.0)

        def sum_body(jj, sv):
            tot = stage_v[0, pl.ds(jj * 16, 16)]
            for rr in range(1, 8):
                tot = tot + stage_v[rr, pl.ds(jj * 16, 16)]
            acc_v[pl.ds(jj * 16, 16)] = tot
            return sv + tot

        sv = lax.fori_loop(0, _NBINS // 16, sum_body,
                           jnp.zeros((16,), jnp.float32))
        ssum = jnp.sum(sv) * inv
        rdenom = 1.0 / jnp.full((16,), ssum + _EPS, dtype=jnp.float32)

        ln2 = jnp.float32(0.6931471805599453)

        def ent_body(jj, tv):
            cnt = acc_v[pl.ds(jj * 16, 16)]
            p = cnt * inv * rdenom
            x = p + _EPS
            i = plsc.bitcast(x, jnp.int32)
            eexp = ((i >> 23) & 0xFF) - 127
            m = plsc.bitcast((i & 0x7FFFFF) | 0x3F800000, jnp.float32)
            r = (m - 1.0) / (m + 1.0)
            t = r * r
            at = r * (1.0 + t * (jnp.float32(1 / 3) + t * (
                jnp.float32(1 / 5) + t * (jnp.float32(1 / 7)
                                          + t * jnp.float32(1 / 9)))))
            lnx = eexp.astype(jnp.float32) * ln2 + 2.0 * at
            return tv + p * lnx

        tv = lax.fori_loop(0, _NBINS // 16, ent_body,
                           jnp.zeros((16,), jnp.float32))
        nsum = -jnp.sum(tv)
        plxv = jnp.exp(jnp.full((16,), nsum, dtype=jnp.float32))
        lanes16 = lax.iota(jnp.int32, 16)
        acc_v[pl.ds(0, 16)] = jnp.where(lanes16 == 0, plxv, 0.0)
        pltpu.sync_copy(acc_v.at[pl.ds(0, 16)],
                        out_hbm.at[pl.ds(gr * 16, 16)])


def _hist_stats(ind):
    mesh = plsc.VectorSubcoreMesh(core_axis_name="c", subcore_axis_name="s")
    k = functools.partial(
        pl.kernel,
        mesh=mesh,
        out_type=jax.ShapeDtypeStruct((4 * 16,), jnp.float32),
        scratch_types=[
            pltpu.VMEM((_CHUNK,), jnp.int32),
            pltpu.VMEM((16 * _NBINS,), jnp.float32),
            pltpu.VMEM((_NBINS,), jnp.float32),
            pltpu.VMEM((8, _NBINS), jnp.float32),
            pltpu.VMEM_SHARED((16, _NBINS), jnp.float32),
        ],
        compiler_params=pltpu.CompilerParams(
            needs_layout_passes=False, skip_device_barrier=True),
    )(_hist_sc_body)
    return k(ind)


def kernel(x, W_in, b_in, W_out, b_out):
    b, d, t = x.shape
    bi = b_in[:, :, None]
    bo = b_out[:, :, None]

    ind_out, qb = _quant_tc(x, W_in, bi)
    feat = _project_tc(qb, W_out, bo)

    perp = _hist_stats(ind_out).reshape(4, 16)[:, 0]

    zeros = jnp.zeros((4,), dtype=x.dtype)
    return zeros, feat, perp, ind_out


# speedup vs baseline: 1.0009x; 1.0009x over previous
import functools

import jax
import jax.numpy as jnp
from jax import lax
from jax.experimental import pallas as pl
from jax.experimental.pallas import tpu as pltpu
from jax.experimental.pallas import tpu_sc as plsc

_G = 2
_R = 2
_DIM = 1024
_DPG = _DIM // _G
_CD = 4
_NIND = 625
_NBINS = 640
_EPS = 1e-5
_HALF_L = 4.0 * (1.0 + 1e-3) / 2.0
_TB = 2048
_TBB = 2048

_NW = 32
_CHUNK = 1024


def _quant_tc_body(x_ref, wi_ref, bi_ref, ind_ref, qb_ref):
    for g in range(_G):
        xb = x_ref[0, g * _DPG:(g + 1) * _DPG]
        h = jnp.dot(wi_ref[g], xb, preferred_element_type=jnp.float32)
        h = h + bi_ref[g]

        def stage(res, scale_inv, scale):
            q = jnp.round(jnp.tanh(res * scale_inv) * _HALF_L)
            idx = ((q[3] * 5.0 + q[2]) * 5.0 + q[1]) * 5.0 + q[0] + 312.0
            return q * (0.5 * scale), idx

        quant0, idx0 = stage(h, 1.0, 1.0)
        quant1, idx1 = stage(h - quant0, 4.0, 0.25)
        qout = quant0 + quant1

        ind_ref[0, g * _R] = idx0.astype(jnp.int32)
        ind_ref[0, g * _R + 1] = idx1.astype(jnp.int32)
        qb_ref[0, g * _CD:(g + 1) * _CD] = qout


def _quant_tc(x, w_in, b_in):
    b, _, t = x.shape
    grid = (b, t // _TB)
    out_shapes = (
        jax.ShapeDtypeStruct((b, _G * _R, t), jnp.int32),
        jax.ShapeDtypeStruct((b, _G * _CD, t), jnp.float32),
    )
    return pl.pallas_call(
        _quant_tc_body,
        grid=grid,
        in_specs=[
            pl.BlockSpec((1, _DIM, _TB), lambda bb, tt: (bb, 0, tt)),
            pl.BlockSpec((_G, _CD, _DPG), lambda bb, tt: (0, 0, 0)),
            pl.BlockSpec((_G, _CD, 1), lambda bb, tt: (0, 0, 0)),
        ],
        out_specs=(
            pl.BlockSpec((1, _G * _R, _TB), lambda bb, tt: (bb, 0, tt)),
            pl.BlockSpec((1, _G * _CD, _TB), lambda bb, tt: (bb, 0, tt)),
        ),
        out_shape=out_shapes,
        compiler_params=pltpu.CompilerParams(
            dimension_semantics=("arbitrary", "arbitrary"),
        ),
    )(x, w_in, b_in)


def _project_tc_body(qb_ref, wo_ref, bo_ref, feat_ref):
    for g in range(_G):
        wo = wo_ref[g]
        wh = wo.astype(jnp.bfloat16)
        wl = (wo - wh.astype(jnp.float32)).astype(jnp.bfloat16)
        qout = qb_ref[0, g * _CD:(g + 1) * _CD].astype(jnp.bfloat16)
        feat_ref[0, g * _DPG:(g + 1) * _DPG] = (
            jnp.dot(wh, qout, preferred_element_type=jnp.float32)
            + jnp.dot(wl, qout, preferred_element_type=jnp.float32)
            + bo_ref[g])


def _project_tc(qb, w_out, b_out3):
    b, _, t = qb.shape
    grid = (b, t // _TBB)
    return pl.pallas_call(
        _project_tc_body,
        grid=grid,
        in_specs=[
            pl.BlockSpec((1, _G * _CD, _TBB), lambda bb, tt: (bb, 0, tt)),
            pl.BlockSpec((_G, _DPG, _CD), lambda bb, tt: (0, 0, 0)),
            pl.BlockSpec((_G, _DPG, 1), lambda bb, tt: (0, 0, 0)),
        ],
        out_specs=pl.BlockSpec((1, _DIM, _TBB), lambda bb, tt: (bb, 0, tt)),
        out_shape=jax.ShapeDtypeStruct((b, _DIM, t), jnp.float32),
        compiler_params=pltpu.CompilerParams(
            dimension_semantics=("arbitrary", "arbitrary"),
        ),
    )(qb, w_out, b_out3)


def _hist_sc_body(ind_hbm, out_hbm, idx_v, hist_v, acc_v, stage_v, shared_v):
    c = lax.axis_index("c")
    s = lax.axis_index("s")
    w = c * 16 + s
    gr = w // 8
    j = w % 8
    b = j // 2
    hh = j % 2

    pltpu.sync_copy(ind_hbm.at[b, gr, pl.ds(hh * _CHUNK, _CHUNK)], idx_v)

    zeros16 = jnp.zeros((16,), dtype=jnp.float32)
    lanes = lax.iota(jnp.int32, 16)

    def zero_body(i, carry):
        for u in range(8):
            hist_v[pl.ds((i * 8 + u) * 16, 16)] = zeros16
        return carry

    lax.fori_loop(0, 16 * _NBINS // 16 // 8, zero_body, 0)

    ones16 = jnp.ones((16,), dtype=jnp.float32)
    lane_off = lanes * _NBINS

    def scat_body(i, carry):
        for u in range(8):
            v = idx_v[pl.ds((i * 8 + u) * 16, 16)]
            plsc.addupdate_scatter(hist_v, [lane_off + v], ones16)
        return carry

    lax.fori_loop(0, _CHUNK // 16 // 8, scat_body, 0)

    def red_body(j, carry):
        tot = hist_v[pl.ds(j * 16, 16)]
        for l in range(1, 16):
            tot = tot + hist_v[pl.ds(l * _NBINS + j * 16, 16)]
        acc_v[pl.ds(j * 16, 16)] = tot
        return carry

    lax.fori_loop(0, _NBINS // 16, red_body, 0)

    pltpu.sync_copy(acc_v, shared_v.at[s])
    plsc.subcore_barrier()

    @pl.when(jnp.logical_or(s == 0, s == 8))
    def _():
        base = (s // 8) * 8
        pltpu.sync_copy(shared_v.at[pl.ds(base, 8)], stage_v)

        inv = jnp.float32(1.0 / 8192.0)

        def sum_body(jj, sv):
            tot = stage_v[0, pl.ds(jj * 16, 16)]
            for rr in range(1, 8):
                tot = tot + stage_v[rr, pl.ds(jj * 16, 16)]
            acc_v[pl.ds(jj * 16, 16)] = tot
            return sv + tot

        sv = lax.fori_loop(0, _NBINS // 16, sum_body,
                           jnp.zeros((16,), jnp.float32))
        ssum = jnp.sum(sv) * inv
        rdenom = 1.0 / jnp.full((16,), ssum + _EPS, dtype=jnp.float32)

        ln2 = jnp.float32(0.6931471805599453)

        def ent_body(jj, tv):
            cnt = acc_v[pl.ds(jj * 16, 16)]
            p = cnt * inv * rdenom
            x = p + _EPS
            i = plsc.bitcast(x, jnp.int32)
            eexp = ((i >> 23) & 0xFF) - 127
            m = plsc.bitcast((i & 0x7FFFFF) | 0x3F800000, jnp.float32)
            r = (m - 1.0) / (m + 1.0)
            t = r * r
            at = r * (1.0 + t * (jnp.float32(1 / 3) + t * (
                jnp.float32(1 / 5) + t * (jnp.float32(1 / 7)
                                          + t * jnp.float32(1 / 9)))))
            lnx = eexp.astype(jnp.float32) * ln2 + 2.0 * at
            return tv + p * lnx

        tv = lax.fori_loop(0, _NBINS // 16, ent_body,
                           jnp.zeros((16,), jnp.float32))
        nsum = -jnp.sum(tv)
        plxv = jnp.exp(jnp.full((16,), nsum, dtype=jnp.float32))
        lanes16 = lax.iota(jnp.int32, 16)
        acc_v[pl.ds(0, 16)] = jnp.where(lanes16 == 0, plxv, 0.0)
        pltpu.sync_copy(acc_v.at[pl.ds(0, 16)],
                        out_hbm.at[pl.ds(gr * 16, 16)])


def _hist_stats(ind):
    mesh = plsc.VectorSubcoreMesh(core_axis_name="c", subcore_axis_name="s")
    k = functools.partial(
        pl.kernel,
        mesh=mesh,
        out_type=jax.ShapeDtypeStruct((4 * 16,), jnp.float32),
        scratch_types=[
            pltpu.VMEM((_CHUNK,), jnp.int32),
            pltpu.VMEM((16 * _NBINS,), jnp.float32),
            pltpu.VMEM((_NBINS,), jnp.float32),
            pltpu.VMEM((8, _NBINS), jnp.float32),
            pltpu.VMEM_SHARED((16, _NBINS), jnp.float32),
        ],
        compiler_params=pltpu.CompilerParams(
            needs_layout_passes=False, skip_device_barrier=True),
    )(_hist_sc_body)
    return k(ind)


def kernel(x, W_in, b_in, W_out, b_out):
    b, d, t = x.shape
    bi = b_in[:, :, None]
    bo = b_out[:, :, None]

    ind_out, qb = _quant_tc(x, W_in, bi)
    feat = _project_tc(qb, W_out, bo)

    perp = _hist_stats(ind_out).reshape(4, 16)[:, 0]

    zeros = jnp.zeros((4,), dtype=x.dtype)
    return zeros, feat, perp, ind_out
